# baseline (speedup 1.0000x reference)
import jax
import jax.numpy as jnp
from jax import lax
from jax.experimental import pallas as pl
from jax.experimental.pallas import tpu as pltpu
from jax.experimental.pallas import tpu_sc as plsc

N = 327680
V = 100000
T = 3
D = 64

NC = 2
NS = 16
NW = NC * NS
L = 16

C = N // NW
CB = 1024
G = C // CB
K = 128
LOG2K = 7
NBK = CB + K


def _body(nt_hbm, ni_hbm, et_hbm, t0_hbm, t1_hbm, t2_hbm, out_hbm,
          tvm, ivm, etf, idxb, metab, av, oc, gsem0, gsem1, wsem, isem):
    wid = lax.axis_index("s") * NC + lax.axis_index("c")

    pltpu.sync_copy(et_hbm, etf)
    lanes = lax.iota(jnp.int32, L)

    def stage(b, buf):
        base = wid * C + b * CB
        pltpu.async_copy(nt_hbm.at[pl.ds(base, CB)], tvm.at[buf], isem)
        pltpu.async_copy(ni_hbm.at[pl.ds(base, CB)], ivm.at[buf], isem)

    stage(0, 0)

    def block(b):
        bb = b & 1
        base = wid * C + b * CB
        pltpu.make_async_copy(nt_hbm.at[pl.ds(base, CB)], tvm.at[bb],
                              isem).wait()
        pltpu.make_async_copy(ni_hbm.at[pl.ds(base, CB)], ivm.at[bb],
                              isem).wait()
        if b + 1 < G:
            stage(b + 1, 1 - bb)

        @plsc.parallel_loop(0, CB // L, carry=(jnp.int32(0), jnp.int32(0),
                                               jnp.int32(0)))
        def compact(i, offs):
            t16 = tvm[bb, pl.ds(i * L, L)]
            d16 = ivm[bb, pl.ds(i * L, L)]
            pair = lax.shift_right_logical(d16, 1)
            meta = (i * L + lanes) * 2 + lax.bitwise_and(d16, 1)
            new_offs = []
            for t in range(T):
                m = t16 == t
                cs = plsc.cumsum(m.astype(jnp.int32))
                o = t * NBK + offs[t] + cs - 1
                plsc.store_scatter(idxb, [o], pair, mask=m)
                plsc.store_scatter(metab, [o], meta, mask=m)
                new_offs.append(offs[t] + jnp.max(cs))
            return tuple(new_offs)

        offs = compact

        zeros16 = jnp.zeros((L,), jnp.int32)
        trash16 = jnp.full((L,), 2 * CB, jnp.int32)

        nbs = []
        for t in range(T):
            off = offs[t]
            nb = lax.shift_right_logical(off + (K - 1), LOG2K)
            padded = lax.shift_left(nb, LOG2K)
            nbs.append(nb)
            for q in range(K // L):
                o = off + q * L + lanes
                m = o < padded
                pad_idx = (wid * 577 + b * 131 + o * 7) & 16383
                plsc.store_scatter(idxb, [t * NBK + o], pad_idx, mask=m)
                plsc.store_scatter(metab, [t * NBK + o], trash16, mask=m)

        nb0, nb1, nb2 = nbs
        n01 = nb0 + nb1
        total = n01 + nb2

        def batch_loc(q):
            tq = (q >= nb0).astype(jnp.int32) + (q >= n01).astype(jnp.int32)
            sq = jnp.where(q < nb0, q * K,
                           jnp.where(q < n01, NBK + (q - nb0) * K,
                                     2 * NBK + (q - n01) * K))
            return tq, sq

        sems = (gsem0, gsem1)
        tbls = (t0_hbm, t1_hbm, t2_hbm)

        def fire(q, buf):
            tq, sq = batch_loc(q)
            for tt in range(T):
                @pl.when(tq == tt)
                def _fire():
                    pltpu.async_copy(tbls[tt].at[idxb.at[pl.ds(sq, K)]],
                                     av.at[buf], sems[buf])

        if b > 0:
            pltpu.make_async_copy(out_hbm.at[pl.ds(0, CB * D)],
                                  oc.at[pl.ds(0, CB * D)], wsem).wait()

        fire(jnp.int32(0), 0)

        def batch(q, _):
            p = lax.bitwise_and(q, 1)
            tq, sq = batch_loc(q)

            for bb in range(2):
                @pl.when(p == bb)
                def _wait():
                    pltpu.make_async_copy(tbls[0].at[idxb.at[pl.ds(0, K)]],
                                          av.at[bb], sems[bb]).wait()

                @pl.when((q + 1 < total) & (p == bb))
                def _fire_next():
                    fire(q + 1, 1 - bb)

            bufs = jnp.full((L,), p, jnp.int32)
            rowi = []
            half = []
            posi = []
            for g in range(K // L):
                meta16 = metab[pl.ds(sq + g * L, L)]
                rowi.append(g * L + lanes)
                half.append(lax.bitwise_and(meta16, 1) * D)
                posi.append(lax.shift_right_logical(meta16, 1) * D)
            etbase = tq * D

            @plsc.parallel_loop(0, D, unroll=2)
            def col(c):
                cr = lax.bitwise_and(c + lanes, D - 1)
                etc = plsc.load_gather(etf, [etbase + cr])
                for g in range(K // L):
                    x = plsc.load_gather(av, [bufs, rowi[g], half[g] + cr])
                    plsc.store_scatter(oc, [posi[g] + cr], x + etc)

            return _

        lax.fori_loop(0, total, batch, 0)

        pltpu.async_copy(oc.at[pl.ds(0, CB * D)],
                         out_hbm.at[pl.ds(base * D, CB * D)], wsem)

    for b in range(G):
        block(b)
    pltpu.make_async_copy(oc.at[pl.ds(0, CB * D)],
                          out_hbm.at[pl.ds(0, CB * D)], wsem).wait()


@jax.jit
def _run(node_type, node_id, embed_type, table0, table1, table2):
    mesh = plsc.VectorSubcoreMesh(core_axis_name="c", subcore_axis_name="s",
                                  num_cores=NC, num_subcores=NS)
    out = pl.kernel(
        _body,
        out_type=jax.ShapeDtypeStruct((N * D,), jnp.float32),
        mesh=mesh,
        compiler_params=pltpu.CompilerParams(needs_layout_passes=False),
        scratch_types=[
            pltpu.VMEM((2, CB), jnp.int32),
            pltpu.VMEM((2, CB), jnp.int32),
            pltpu.VMEM((T * D,), jnp.float32),
            pltpu.VMEM((T * NBK,), jnp.int32),
            pltpu.VMEM((T * NBK,), jnp.int32),
            pltpu.VMEM((2, K, 2 * D), jnp.float32),
            pltpu.VMEM(((CB + 1) * D,), jnp.float32),
            pltpu.SemaphoreType.DMA,
            pltpu.SemaphoreType.DMA,
            pltpu.SemaphoreType.DMA,
            pltpu.SemaphoreType.DMA,
        ],
    )(node_type, node_id, embed_type.reshape(T * D),
      table0.reshape(V // 2, 2 * D), table1.reshape(V // 2, 2 * D),
      table2.reshape(V // 2, 2 * D))
    return out.reshape(N, D)


def kernel(node_type, node_id, embed_type, table0, table1, table2):
    return _run(node_type.astype(jnp.int32), node_id.astype(jnp.int32),
                embed_type, table0, table1, table2)


# speedup vs baseline: 1.0071x; 1.0071x over previous
import jax
import jax.numpy as jnp
from jax import lax
from jax.experimental import pallas as pl
from jax.experimental.pallas import tpu as pltpu
from jax.experimental.pallas import tpu_sc as plsc

N = 327680
V = 100000
T = 3
D = 64

NC = 2
NS = 16
NW = NC * NS
L = 16

C = N // NW
CB = 1024
G = C // CB
K = 128
LOG2K = 7
NBK = CB + K


def _body(nt_hbm, ni_hbm, et_hbm, t0_hbm, t1_hbm, t2_hbm, out_hbm,
          tvm, ivm, etf, idxb, metab, av, oc, gsem0, gsem1, wsem):
    wid = lax.axis_index("s") * NC + lax.axis_index("c")

    pltpu.sync_copy(et_hbm, etf)
    lanes = lax.iota(jnp.int32, L)

    def block(b, _):
        base = wid * C + b * CB
        pltpu.sync_copy(nt_hbm.at[pl.ds(base, CB)], tvm)
        pltpu.sync_copy(ni_hbm.at[pl.ds(base, CB)], ivm)

        @plsc.parallel_loop(0, CB // L, carry=(jnp.int32(0), jnp.int32(0),
                                               jnp.int32(0)))
        def compact(i, offs):
            t16 = tvm[pl.ds(i * L, L)]
            d16 = ivm[pl.ds(i * L, L)]
            pair = lax.shift_right_logical(d16, 1)
            meta = (i * L + lanes) * 2 + lax.bitwise_and(d16, 1)
            new_offs = []
            for t in range(T):
                m = t16 == t
                cs = plsc.cumsum(m.astype(jnp.int32))
                o = t * NBK + offs[t] + cs - 1
                plsc.store_scatter(idxb, [o], pair, mask=m)
                plsc.store_scatter(metab, [o], meta, mask=m)
                new_offs.append(offs[t] + jnp.max(cs))
            return tuple(new_offs)

        offs = compact

        zeros16 = jnp.zeros((L,), jnp.int32)
        trash16 = jnp.full((L,), 2 * CB, jnp.int32)

        nbs = []
        for t in range(T):
            off = offs[t]
            nb = lax.shift_right_logical(off + (K - 1), LOG2K)
            padded = lax.shift_left(nb, LOG2K)
            nbs.append(nb)
            for q in range(K // L):
                o = off + q * L + lanes
                m = o < padded
                pad_idx = (wid * 577 + b * 131 + o * 7) & 16383
                plsc.store_scatter(idxb, [t * NBK + o], pad_idx, mask=m)
                plsc.store_scatter(metab, [t * NBK + o], trash16, mask=m)

        nb0, nb1, nb2 = nbs
        n01 = nb0 + nb1
        total = n01 + nb2

        def batch_loc(q):
            tq = (q >= nb0).astype(jnp.int32) + (q >= n01).astype(jnp.int32)
            sq = jnp.where(q < nb0, q * K,
                           jnp.where(q < n01, NBK + (q - nb0) * K,
                                     2 * NBK + (q - n01) * K))
            return tq, sq

        sems = (gsem0, gsem1)
        tbls = (t0_hbm, t1_hbm, t2_hbm)

        def fire(q, buf):
            tq, sq = batch_loc(q)
            for tt in range(T):
                @pl.when(tq == tt)
                def _fire():
                    pltpu.async_copy(tbls[tt].at[idxb.at[pl.ds(sq, K)]],
                                     av.at[buf], sems[buf])

        @pl.when(b > 0)
        def _wait_wb():
            pltpu.make_async_copy(out_hbm.at[pl.ds(0, CB * D)],
                                  oc.at[pl.ds(0, CB * D)], wsem).wait()

        fire(jnp.int32(0), 0)

        def batch(q, _):
            p = lax.bitwise_and(q, 1)
            tq, sq = batch_loc(q)

            for bb in range(2):
                @pl.when(p == bb)
                def _wait():
                    pltpu.make_async_copy(tbls[0].at[idxb.at[pl.ds(0, K)]],
                                          av.at[bb], sems[bb]).wait()

                @pl.when((q + 1 < total) & (p == bb))
                def _fire_next():
                    fire(q + 1, 1 - bb)

            bufs = jnp.full((L,), p, jnp.int32)
            rowi = []
            half = []
            posi = []
            for g in range(K // L):
                meta16 = metab[pl.ds(sq + g * L, L)]
                rowi.append(g * L + lanes)
                half.append(lax.bitwise_and(meta16, 1) * D)
                posi.append(lax.shift_right_logical(meta16, 1) * D)
            etbase = tq * D

            @plsc.parallel_loop(0, D, unroll=2)
            def col(c):
                cr = lax.bitwise_and(c + lanes, D - 1)
                etc = plsc.load_gather(etf, [etbase + cr])
                for g in range(K // L):
                    x = plsc.load_gather(av, [bufs, rowi[g], half[g] + cr])
                    plsc.store_scatter(oc, [posi[g] + cr], x + etc)

            return _

        lax.fori_loop(0, total, batch, 0)

        pltpu.async_copy(oc.at[pl.ds(0, CB * D)],
                         out_hbm.at[pl.ds(base * D, CB * D)], wsem)
        return _

    lax.fori_loop(0, G, block, 0)
    pltpu.make_async_copy(oc.at[pl.ds(0, CB * D)],
                          out_hbm.at[pl.ds(0, CB * D)], wsem).wait()


@jax.jit
def _run(node_type, node_id, embed_type, table0, table1, table2):
    mesh = plsc.VectorSubcoreMesh(core_axis_name="c", subcore_axis_name="s",
                                  num_cores=NC, num_subcores=NS)
    out = pl.kernel(
        _body,
        out_type=jax.ShapeDtypeStruct((N * D,), jnp.float32),
        mesh=mesh,
        compiler_params=pltpu.CompilerParams(needs_layout_passes=False),
        scratch_types=[
            pltpu.VMEM((CB,), jnp.int32),
            pltpu.VMEM((CB,), jnp.int32),
            pltpu.VMEM((T * D,), jnp.float32),
            pltpu.VMEM((T * NBK,), jnp.int32),
            pltpu.VMEM((T * NBK,), jnp.int32),
            pltpu.VMEM((2, K, 2 * D), jnp.float32),
            pltpu.VMEM(((CB + 1) * D,), jnp.float32),
            pltpu.SemaphoreType.DMA,
            pltpu.SemaphoreType.DMA,
            pltpu.SemaphoreType.DMA,
        ],
    )(node_type, node_id, embed_type.reshape(T * D),
      table0.reshape(V // 2, 2 * D), table1.reshape(V // 2, 2 * D),
      table2.reshape(V // 2, 2 * D))
    return out.reshape(N, D)


def kernel(node_type, node_id, embed_type, table0, table1, table2):
    return _run(node_type.astype(jnp.int32), node_id.astype(jnp.int32),
                embed_type, table0, table1, table2)
